# Initial kernel scaffold; baseline (speedup 1.0000x reference)
import jax
import jax.numpy as jnp
from jax.experimental import pallas as pl


def kernel(tokens, token_table, style_embedding):
    raise NotImplementedError("write your pallas kernel here")


# speedup vs baseline: 1.1722x; 1.1722x over previous
import jax
import jax.numpy as jnp
from jax import lax
from jax.experimental import pallas as pl
from jax.experimental.pallas import tpu as pltpu
from jax.experimental.pallas import tpu_sc as plsc

VOCAB = 49408
D = 512
SEQ = 77
SEQ_PAD = 80
N_CLS = 345
N_STYLE = 8
NC, NS = 2, 16
NW = NC * NS


def _body(tokens_hbm, table_hbm, style_hbm, out_hbm, tok_row, buf, styles_v, sem):
    wid = lax.axis_index("s") * NC + lax.axis_index("c")
    c0 = wid * N_CLS // NW
    c1 = (wid + 1) * N_CLS // NW
    pltpu.sync_copy(style_hbm, styles_v)

    def per_class(i, _):
        c = c0 + i
        pltpu.sync_copy(tokens_hbm.at[c], tok_row)
        pltpu.async_copy(table_hbm.at[tok_row], buf, sem).wait()
        for s in range(N_STYLE):
            row = s * N_CLS + c
            pltpu.sync_copy(buf.at[pl.ds(0, 2)], out_hbm.at[row, pl.ds(0, 2)])
            pltpu.sync_copy(styles_v.at[s], out_hbm.at[row, 2])
            pltpu.sync_copy(buf.at[pl.ds(3, SEQ - 3)], out_hbm.at[row, pl.ds(3, SEQ - 3)])
        return ()

    lax.fori_loop(0, c1 - c0, per_class, ())


def kernel(tokens, token_table, style_embedding):
    tokens_pad = jnp.pad(tokens, ((0, 0), (0, SEQ_PAD - SEQ)))
    styles = style_embedding.reshape(N_STYLE, D)
    k = pl.kernel(
        _body,
        out_type=jax.ShapeDtypeStruct((N_STYLE * N_CLS, SEQ, D), jnp.float32),
        mesh=plsc.VectorSubcoreMesh(
            core_axis_name="c", subcore_axis_name="s", num_cores=NC, num_subcores=NS
        ),
        scratch_types=[
            pltpu.VMEM((SEQ_PAD,), jnp.int32),
            pltpu.VMEM((SEQ_PAD, D), jnp.float32),
            pltpu.VMEM((N_STYLE, D), jnp.float32),
            pltpu.SemaphoreType.DMA,
        ],
        compiler_params=pltpu.CompilerParams(use_tc_tiling_on_sc=False),
    )
    return k(tokens_pad, token_table, styles)
